# baseline (speedup 1.0000x reference)
import functools

import jax
import jax.numpy as jnp
from jax.experimental import pallas as pl
from jax.experimental.pallas import tpu as pltpu
from jax.experimental.pallas import tpu_sc as plsc

_K = 1024
_D = 64
_TILE = 512
_CH = 96


def _vq_idx_tile(x_ref, cb2_ref, csq_ref, idx_ref):
    x = x_ref[...]
    xsq = jnp.sum(x * x, axis=1, keepdims=True)
    m2 = jax.lax.dot_general(x, cb2_ref[...],
                             (((1,), (1,)), ((), ())),
                             preferred_element_type=jnp.float32)
    dist = (xsq + csq_ref[...]) - m2
    minval = jnp.min(dist, axis=1, keepdims=True)
    iota = jax.lax.broadcasted_iota(jnp.int32, dist.shape, 1)
    idx = jnp.min(jnp.where(dist == minval, iota, _K), axis=1)
    idx_ref[...] = idx.reshape(1, _TILE // 128, 128)


def _tc_argmin(flat, codebook):
    n = flat.shape[0]
    csq = jnp.sum(codebook ** 2, axis=1)[None, :]
    cb2 = codebook + codebook
    return pl.pallas_call(
        _vq_idx_tile,
        grid=(n // _TILE,),
        in_specs=[
            pl.BlockSpec((_TILE, _D), lambda i: (i, 0)),
            pl.BlockSpec((_K, _D), lambda i: (0, 0)),
            pl.BlockSpec((1, _K), lambda i: (0, 0)),
        ],
        out_specs=pl.BlockSpec((1, _TILE // 128, 128), lambda i: (i, 0, 0)),
        out_shape=jax.ShapeDtypeStruct((n // _TILE, _TILE // 128, 128),
                                       jnp.int32),
        compiler_params=pltpu.CompilerParams(
            dimension_semantics=("parallel",)),
    )(flat, cb2, csq)


def _sc_gather(codebook, idx, n):
    cb_pad = jnp.pad(codebook, ((0, 0), (0, 128 - _D)))
    info = plsc.get_sparse_core_info()
    nw = info.num_cores * info.num_subcores
    b_per_w = n // nw
    n_ch = b_per_w // _CH
    idx3d = idx.reshape(nw, n_ch, _CH)
    mesh = plsc.VectorSubcoreMesh(core_axis_name="c", subcore_axis_name="s")

    @functools.partial(
        pl.kernel, mesh=mesh,
        out_type=jax.ShapeDtypeStruct((n, 128), jnp.float32),
        scratch_types=[
            pltpu.VMEM((n_ch, _CH), jnp.int32),
            pltpu.VMEM((b_per_w, 128), jnp.float32),
            pltpu.SemaphoreType.DMA,
        ],
    )
    def gather_k(cb_hbm, idx_hbm, out_hbm, idx_v, rows_v, sem):
        wid = jax.lax.axis_index("s") * info.num_cores + jax.lax.axis_index("c")
        pltpu.sync_copy(idx_hbm.at[wid], idx_v)
        copies = [
            pltpu.async_copy(cb_hbm.at[idx_v.at[j]],
                             rows_v.at[pl.ds(j * _CH, _CH)], sem)
            for j in range(n_ch)
        ]
        for c in copies:
            c.wait()
        pltpu.sync_copy(rows_v, out_hbm.at[pl.ds(wid * b_per_w, b_per_w)])

    return gather_k(cb_pad, idx3d)[:, :_D]


def kernel(inputs, codebook):
    input_shape = inputs.shape
    flat = inputs.reshape(-1, _D)
    n = flat.shape[0]
    idx = _tc_argmin(flat, codebook)
    q = _sc_gather(codebook, idx.reshape(-1), n)
    quantized = (flat + (q - flat)).reshape(input_shape)
    return (quantized, idx.reshape(-1))


# speedup vs baseline: 1.1325x; 1.1325x over previous
import functools

import jax
import jax.numpy as jnp
from jax.experimental import pallas as pl
from jax.experimental.pallas import tpu as pltpu
from jax.experimental.pallas import tpu_sc as plsc

_K = 1024
_D = 64
_TILE = 576
_CH = 96


def _vq_idx_tile(x_ref, cb2_ref, csq_ref, idx_ref):
    x = x_ref[0]
    xsq = jnp.sum(x * x, axis=1, keepdims=True)
    m2 = jax.lax.dot_general(x, cb2_ref[...],
                             (((1,), (1,)), ((), ())),
                             preferred_element_type=jnp.float32)
    dist = (xsq + csq_ref[...]) - m2
    minval = jnp.min(dist, axis=1, keepdims=True)
    iota_f = jax.lax.broadcasted_iota(jnp.int32, dist.shape, 1).astype(
        jnp.float32)
    idx = jnp.min(jnp.where(dist == minval, iota_f, float(_K)),
                  axis=1).astype(jnp.int32)
    idx_ref[...] = idx.reshape(1, _TILE // _CH, _CH)


def _tc_argmin(inputs3d, codebook):
    b = inputs3d.shape[0]
    csq = jnp.sum(codebook ** 2, axis=1)[None, :]
    cb2 = codebook + codebook
    return pl.pallas_call(
        _vq_idx_tile,
        grid=(b,),
        in_specs=[
            pl.BlockSpec((1, _TILE, _D), lambda i: (i, 0, 0)),
            pl.BlockSpec((_K, _D), lambda i: (0, 0)),
            pl.BlockSpec((1, _K), lambda i: (0, 0)),
        ],
        out_specs=pl.BlockSpec((1, _TILE // _CH, _CH), lambda i: (i, 0, 0)),
        out_shape=jax.ShapeDtypeStruct((b, _TILE // _CH, _CH), jnp.int32),
        compiler_params=pltpu.CompilerParams(
            dimension_semantics=("parallel",)),
    )(inputs3d, cb2, csq)


def _sc_gather(codebook, idx3d, n):
    cb_pad = jnp.pad(codebook, ((0, 0), (0, 128 - _D)))
    info = plsc.get_sparse_core_info()
    nw = info.num_cores * info.num_subcores
    b_per_w = n // nw
    n_ch = b_per_w // _CH
    assert idx3d.shape == (nw, n_ch, _CH)
    mesh = plsc.VectorSubcoreMesh(core_axis_name="c", subcore_axis_name="s")

    @functools.partial(
        pl.kernel, mesh=mesh,
        out_type=jax.ShapeDtypeStruct((n, 128), jnp.float32),
        scratch_types=[
            pltpu.VMEM((n_ch, _CH), jnp.int32),
            pltpu.VMEM((b_per_w, 128), jnp.float32),
            pltpu.SemaphoreType.DMA,
        ],
    )
    def gather_k(cb_hbm, idx_hbm, out_hbm, idx_v, rows_v, sem):
        wid = jax.lax.axis_index("s") * info.num_cores + jax.lax.axis_index("c")
        pltpu.sync_copy(idx_hbm.at[wid], idx_v)
        copies = [
            pltpu.async_copy(cb_hbm.at[idx_v.at[j]],
                             rows_v.at[pl.ds(j * _CH, _CH)], sem)
            for j in range(n_ch)
        ]
        for c in copies:
            c.wait()
        pltpu.sync_copy(rows_v, out_hbm.at[pl.ds(wid * b_per_w, b_per_w)])

    return gather_k(cb_pad, idx3d)[:, :_D]


def kernel(inputs, codebook):
    input_shape = inputs.shape
    n = input_shape[0] * input_shape[1]
    idx3d = _tc_argmin(inputs, codebook)
    q = _sc_gather(codebook, idx3d, n)
    quantized = inputs + (q.reshape(input_shape) - inputs)
    return (quantized, idx3d.reshape(-1))


# speedup vs baseline: 1.2612x; 1.1137x over previous
import functools

import jax
import jax.numpy as jnp
from jax.experimental import pallas as pl
from jax.experimental.pallas import tpu as pltpu
from jax.experimental.pallas import tpu_sc as plsc

_K = 1024
_D = 64
_TILE = 576
_CH = 96


def _vq_idx_tile(xt_hbm, cb2t_ref, csqt_ref, idx_ref, xbuf, sem):
    i = pl.program_id(0)
    nb = pl.num_programs(0)
    slot = jax.lax.rem(i, 2)
    nxt = jax.lax.rem(i + 1, 2)

    @pl.when(i == 0)
    def _():
        pltpu.make_async_copy(xt_hbm.at[0], xbuf.at[0], sem.at[0]).start()

    @pl.when(i + 1 < nb)
    def _():
        pltpu.make_async_copy(xt_hbm.at[i + 1], xbuf.at[nxt],
                              sem.at[nxt]).start()

    pltpu.make_async_copy(xt_hbm.at[i], xbuf.at[slot], sem.at[slot]).wait()
    xt = xbuf[slot]
    xsq = jnp.sum(xt * xt, axis=0, keepdims=True)
    m2 = jax.lax.dot_general(cb2t_ref[...], xt,
                             (((0,), (0,)), ((), ())),
                             preferred_element_type=jnp.float32)
    dist = (xsq + csqt_ref[...]) - m2
    minval = jnp.min(dist, axis=0, keepdims=True)
    iota_f = jax.lax.broadcasted_iota(jnp.int32, dist.shape, 0).astype(
        jnp.float32)
    idx = jnp.min(jnp.where(dist == minval, iota_f, float(_K)), axis=0,
                  keepdims=True).astype(jnp.int32)
    idx_ref[...] = idx.reshape(1, 1, _TILE)


def _tc_argmin(inputs3d, codebook):
    b = inputs3d.shape[0]
    xt = jnp.transpose(inputs3d, (0, 2, 1))
    cb2t = codebook.T + codebook.T
    csqt = jnp.sum(codebook ** 2, axis=1)[:, None]
    return pl.pallas_call(
        _vq_idx_tile,
        grid=(b,),
        in_specs=[
            pl.BlockSpec(memory_space=pltpu.MemorySpace.HBM),
            pl.BlockSpec((_D, _K), lambda i: (0, 0)),
            pl.BlockSpec((_K, 1), lambda i: (0, 0)),
        ],
        out_specs=pl.BlockSpec((1, 1, _TILE), lambda i: (i, 0, 0)),
        out_shape=jax.ShapeDtypeStruct((b, 1, _TILE), jnp.int32),
        scratch_shapes=[
            pltpu.VMEM((2, _D, _TILE), jnp.float32),
            pltpu.SemaphoreType.DMA((2,)),
        ],
        compiler_params=pltpu.CompilerParams(
            dimension_semantics=("arbitrary",)),
    )(xt, cb2t, csqt)


def _sc_gather(codebook, idx3d, n):
    cb_pad = jnp.pad(codebook, ((0, 0), (0, 128 - _D)))
    info = plsc.get_sparse_core_info()
    nw = info.num_cores * info.num_subcores
    b_per_w = n // nw
    n_ch = b_per_w // _CH
    assert idx3d.shape == (nw, n_ch, _CH)
    mesh = plsc.VectorSubcoreMesh(core_axis_name="c", subcore_axis_name="s")

    @functools.partial(
        pl.kernel, mesh=mesh,
        out_type=jax.ShapeDtypeStruct((n, 128), jnp.float32),
        scratch_types=[
            pltpu.VMEM((n_ch, _CH), jnp.int32),
            pltpu.VMEM((b_per_w, 128), jnp.float32),
            pltpu.SemaphoreType.DMA,
        ],
    )
    def gather_k(cb_hbm, idx_hbm, out_hbm, idx_v, rows_v, sem):
        wid = jax.lax.axis_index("s") * info.num_cores + jax.lax.axis_index("c")
        pltpu.sync_copy(idx_hbm.at[wid], idx_v)
        copies = [
            pltpu.async_copy(cb_hbm.at[idx_v.at[j]],
                             rows_v.at[pl.ds(j * _CH, _CH)], sem)
            for j in range(n_ch)
        ]
        for c in copies:
            c.wait()
        pltpu.sync_copy(rows_v, out_hbm.at[pl.ds(wid * b_per_w, b_per_w)])

    return gather_k(cb_pad, idx3d)[:, :_D]


def kernel(inputs, codebook):
    input_shape = inputs.shape
    n = input_shape[0] * input_shape[1]
    idx = _tc_argmin(inputs, codebook)
    nw = input_shape[0]
    idx3d = idx.reshape(nw, _TILE // _CH, _CH)
    q = _sc_gather(codebook, idx3d, n)
    return (q.reshape(input_shape), idx.reshape(-1))


# speedup vs baseline: 1.3451x; 1.0665x over previous
import functools

import jax
import jax.numpy as jnp
from jax.experimental import pallas as pl
from jax.experimental.pallas import tpu as pltpu
from jax.experimental.pallas import tpu_sc as plsc

_K = 1024
_D = 64
_TILE = 576
_CH = 96


def _vq_idx_tile(xt_hbm, cb2t_ref, csqt_ref, iota_ref, idx_ref, xbuf, sem):
    i = pl.program_id(0)
    nb = pl.num_programs(0)
    slot = jax.lax.rem(i, 2)
    nxt = jax.lax.rem(i + 1, 2)

    @pl.when(i == 0)
    def _():
        pltpu.make_async_copy(xt_hbm.at[0], xbuf.at[0], sem.at[0]).start()

    @pl.when(i + 1 < nb)
    def _():
        pltpu.make_async_copy(xt_hbm.at[i + 1], xbuf.at[nxt],
                              sem.at[nxt]).start()

    pltpu.make_async_copy(xt_hbm.at[i], xbuf.at[slot], sem.at[slot]).wait()
    xt = xbuf[slot]
    xsq = jnp.sum(xt * xt, axis=0, keepdims=True)
    m2 = jax.lax.dot_general(cb2t_ref[...], xt,
                             (((0,), (0,)), ((), ())),
                             preferred_element_type=jnp.float32)
    dist = (xsq + csqt_ref[...]) - m2
    minval = jnp.min(dist, axis=0, keepdims=True)
    idx = jnp.min(jnp.where(dist == minval, iota_ref[...], float(_K)),
                  axis=0, keepdims=True).astype(jnp.int32)
    idx_ref[...] = idx.reshape(1, 1, _TILE)


def _tc_argmin(inputs3d, codebook):
    b = inputs3d.shape[0]
    xt = jnp.transpose(inputs3d, (0, 2, 1))
    cb2t = codebook.T + codebook.T
    csqt = jnp.sum(codebook ** 2, axis=1)[:, None]
    iota_col = jnp.arange(_K, dtype=jnp.float32)[:, None]
    return pl.pallas_call(
        _vq_idx_tile,
        grid=(b,),
        in_specs=[
            pl.BlockSpec(memory_space=pltpu.MemorySpace.HBM),
            pl.BlockSpec((_D, _K), lambda i: (0, 0)),
            pl.BlockSpec((_K, 1), lambda i: (0, 0)),
            pl.BlockSpec((_K, 1), lambda i: (0, 0)),
        ],
        out_specs=pl.BlockSpec((1, 1, _TILE), lambda i: (i, 0, 0)),
        out_shape=jax.ShapeDtypeStruct((b, 1, _TILE), jnp.int32),
        scratch_shapes=[
            pltpu.VMEM((2, _D, _TILE), jnp.float32),
            pltpu.SemaphoreType.DMA((2,)),
        ],
        compiler_params=pltpu.CompilerParams(
            dimension_semantics=("arbitrary",)),
    )(xt, cb2t, csqt, iota_col)


def _sc_gather(codebook, idx3d, n):
    cb_pad = jnp.pad(codebook, ((0, 0), (0, 128 - _D)))
    info = plsc.get_sparse_core_info()
    nw = info.num_cores * info.num_subcores
    b_per_w = n // nw
    n_ch = b_per_w // _CH
    assert idx3d.shape == (nw, n_ch, _CH)
    mesh = plsc.VectorSubcoreMesh(core_axis_name="c", subcore_axis_name="s")

    @functools.partial(
        pl.kernel, mesh=mesh,
        out_type=jax.ShapeDtypeStruct((n, 128), jnp.float32),
        scratch_types=[
            pltpu.VMEM((n_ch, _CH), jnp.int32),
            pltpu.VMEM((b_per_w, 128), jnp.float32),
            pltpu.VMEM_SHARED((_K, 128), jnp.float32),
            pltpu.SemaphoreType.DMA,
        ],
    )
    def gather_k(cb_hbm, idx_hbm, out_hbm, idx_v, rows_v, cb_sh, sem):
        sid = jax.lax.axis_index("s")
        wid = sid * info.num_cores + jax.lax.axis_index("c")

        @pl.when(sid == 0)
        def _():
            pltpu.sync_copy(cb_hbm, cb_sh)

        pltpu.sync_copy(idx_hbm.at[wid], idx_v)
        plsc.subcore_barrier()
        copies = [
            pltpu.async_copy(cb_sh.at[idx_v.at[j]],
                             rows_v.at[pl.ds(j * _CH, _CH)], sem)
            for j in range(n_ch)
        ]
        for c in copies:
            c.wait()
        pltpu.sync_copy(rows_v, out_hbm.at[pl.ds(wid * b_per_w, b_per_w)])

    return gather_k(cb_pad, idx3d)[:, :_D]


def kernel(inputs, codebook):
    input_shape = inputs.shape
    n = input_shape[0] * input_shape[1]
    idx = _tc_argmin(inputs, codebook)
    nw = input_shape[0]
    idx3d = idx.reshape(nw, _TILE // _CH, _CH)
    q = _sc_gather(codebook, idx3d, n)
    return (q.reshape(input_shape), idx.reshape(-1))


# speedup vs baseline: 1.5084x; 1.1214x over previous
import functools

import jax
import jax.numpy as jnp
from jax.experimental import pallas as pl
from jax.experimental.pallas import tpu as pltpu
from jax.experimental.pallas import tpu_sc as plsc

_K = 1024
_D = 64
_TILE = 576
_CH = 96


_ROWS = 2


def _vq_idx_tile(xt_hbm, cb2t_ref, csqt_ref, idx_ref, xbuf, sem):
    i = pl.program_id(0)
    nb = pl.num_programs(0)
    slot = jax.lax.rem(i, 2)
    nxt = jax.lax.rem(i + 1, 2)

    @pl.when(i == 0)
    def _():
        pltpu.make_async_copy(xt_hbm.at[pl.ds(0, _ROWS)], xbuf.at[0],
                              sem.at[0]).start()

    @pl.when(i + 1 < nb)
    def _():
        pltpu.make_async_copy(xt_hbm.at[pl.ds((i + 1) * _ROWS, _ROWS)],
                              xbuf.at[nxt], sem.at[nxt]).start()

    pltpu.make_async_copy(xt_hbm.at[pl.ds(i * _ROWS, _ROWS)], xbuf.at[slot],
                          sem.at[slot]).wait()
    xt = jnp.concatenate([xbuf[slot, r] for r in range(_ROWS)],
                         axis=1)
    xsq = jnp.sum(xt * xt, axis=0, keepdims=True)
    m2 = jax.lax.dot_general(cb2t_ref[...], xt,
                             (((0,), (0,)), ((), ())),
                             preferred_element_type=jnp.float32)
    dist = (xsq + csqt_ref[...]) - m2
    minval = jnp.min(dist, axis=0, keepdims=True)
    iota_col = jax.lax.broadcasted_iota(jnp.int32, (_K, 1), 0).astype(
        jnp.float32)
    idx = jnp.min(jnp.where(dist == minval, iota_col, float(_K)),
                  axis=0, keepdims=True).astype(jnp.int32)
    idx_ref[...] = idx.reshape(1, 1, _ROWS * _TILE)


def _tc_argmin(inputs3d, codebook):
    b = inputs3d.shape[0]
    xt = jnp.transpose(inputs3d, (0, 2, 1))
    cb2t = codebook.T + codebook.T
    csqt = jnp.sum(codebook ** 2, axis=1)[:, None]
    return pl.pallas_call(
        _vq_idx_tile,
        grid=(b // _ROWS,),
        in_specs=[
            pl.BlockSpec(memory_space=pltpu.MemorySpace.HBM),
            pl.BlockSpec((_D, _K), lambda i: (0, 0)),
            pl.BlockSpec((_K, 1), lambda i: (0, 0)),
        ],
        out_specs=pl.BlockSpec((1, 1, _ROWS * _TILE), lambda i: (i, 0, 0)),
        out_shape=jax.ShapeDtypeStruct((b // _ROWS, 1, _ROWS * _TILE),
                                       jnp.int32),
        scratch_shapes=[
            pltpu.VMEM((2, _ROWS, _D, _TILE), jnp.float32),
            pltpu.SemaphoreType.DMA((2,)),
        ],
        compiler_params=pltpu.CompilerParams(
            dimension_semantics=("arbitrary",)),
    )(xt, cb2t, csqt)


def _sc_gather(codebook, idx3d, n):
    cb_pad = jnp.pad(codebook, ((0, 0), (0, 128 - _D)))
    info = plsc.get_sparse_core_info()
    nw = info.num_cores * info.num_subcores
    b_per_w = n // nw
    n_ch = b_per_w // _CH
    assert idx3d.shape == (nw, n_ch, _CH)
    mesh = plsc.VectorSubcoreMesh(core_axis_name="c", subcore_axis_name="s")

    @functools.partial(
        pl.kernel, mesh=mesh,
        out_type=jax.ShapeDtypeStruct((n, 128), jnp.float32),
        scratch_types=[
            pltpu.VMEM((n_ch, _CH), jnp.int32),
            pltpu.VMEM((b_per_w, 128), jnp.float32),
            pltpu.VMEM_SHARED((_K, 128), jnp.float32),
            pltpu.SemaphoreType.DMA,
        ],
    )
    def gather_k(cb_hbm, idx_hbm, out_hbm, idx_v, rows_v, cb_sh, sem):
        sid = jax.lax.axis_index("s")
        wid = sid * info.num_cores + jax.lax.axis_index("c")

        @pl.when(sid == 0)
        def _():
            pltpu.sync_copy(cb_hbm, cb_sh)

        pltpu.sync_copy(idx_hbm.at[wid], idx_v)
        plsc.subcore_barrier()
        copies = [
            pltpu.async_copy(cb_sh.at[idx_v.at[j]],
                             rows_v.at[pl.ds(j * _CH, _CH)], sem)
            for j in range(n_ch)
        ]
        for c in copies:
            c.wait()
        pltpu.sync_copy(rows_v, out_hbm.at[pl.ds(wid * b_per_w, b_per_w)])

    return gather_k(cb_pad, idx3d)[:, :_D]


def kernel(inputs, codebook):
    input_shape = inputs.shape
    n = input_shape[0] * input_shape[1]
    idx = _tc_argmin(inputs, codebook)
    nw = input_shape[0]
    idx3d = idx.reshape(nw, _TILE // _CH, _CH)
    q = _sc_gather(codebook, idx3d, n)
    return (q.reshape(input_shape), idx.reshape(-1))
